# baseline (speedup 1.0000x reference)
import functools

import jax
import jax.numpy as jnp
from jax import lax
from jax.experimental import pallas as pl
from jax.experimental.pallas import tpu as pltpu
from jax.experimental.pallas import tpu_sc as plsc

_CHUNK = 128


@functools.lru_cache(maxsize=None)
def _build(batch, emb, n_users, n_items):
    info = plsc.get_sparse_core_info()
    nw = info.num_cores * info.num_subcores
    b_per_w = batch // nw
    assert b_per_w % _CHUNK == 0
    ch = b_per_w // _CHUNK

    mesh = plsc.VectorSubcoreMesh(core_axis_name="c", subcore_axis_name="s")

    @functools.partial(
        pl.kernel,
        mesh=mesh,
        compiler_params=pltpu.CompilerParams(use_tc_tiling_on_sc=False),
        out_type=(
            jax.ShapeDtypeStruct((batch, emb), jnp.float32),
            jax.ShapeDtypeStruct((batch, emb), jnp.float32),
        ),
        scratch_types=[
            pltpu.VMEM((ch, _CHUNK), jnp.int32),
            pltpu.VMEM((ch, _CHUNK), jnp.int32),
            pltpu.VMEM((b_per_w, emb), jnp.float32),
            pltpu.VMEM((b_per_w, emb), jnp.float32),
            pltpu.SemaphoreType.DMA,
            pltpu.SemaphoreType.DMA,
        ],
    )
    def k(user_hbm, item_hbm, ut_hbm, it_hbm, uout, iout,
          uidx, iidx, urows, irows, usem, isem):
        wid = lax.axis_index("s") * info.num_cores + lax.axis_index("c")
        base = wid * b_per_w
        pltpu.sync_copy(user_hbm.at[wid], uidx)
        pltpu.sync_copy(item_hbm.at[wid], iidx)
        copies = []
        for j in range(ch):
            copies.append(pltpu.async_copy(
                ut_hbm.at[uidx.at[j]],
                urows.at[pl.ds(j * _CHUNK, _CHUNK)], usem))
            copies.append(pltpu.async_copy(
                it_hbm.at[iidx.at[j]],
                irows.at[pl.ds(j * _CHUNK, _CHUNK)], isem))
        for cp in copies:
            cp.wait()
        pltpu.sync_copy(urows, uout.at[pl.ds(base, b_per_w)])
        pltpu.sync_copy(irows, iout.at[pl.ds(base, b_per_w)])

    return k, nw, ch


def kernel(user, item, user_table, item_table):
    batch, = user.shape
    n_users, emb = user_table.shape
    n_items, _ = item_table.shape
    k, nw, ch = _build(batch, emb, n_users, n_items)
    user3 = user.reshape(nw, ch, _CHUNK)
    item3 = item.reshape(nw, ch, _CHUNK)
    return k(user3, item3, user_table, item_table)


# speedup vs baseline: 1.5655x; 1.5655x over previous
import functools

import jax
import jax.numpy as jnp
from jax import lax
from jax.experimental import pallas as pl
from jax.experimental.pallas import tpu as pltpu
from jax.experimental.pallas import tpu_sc as plsc

_CH = 128


@functools.lru_cache(maxsize=None)
def _build(batch, emb, n_users, n_items):
    info = plsc.get_sparse_core_info()
    nw = info.num_cores * info.num_subcores
    lanes = info.num_lanes
    b_per_w = batch // nw
    assert b_per_w * nw == batch and b_per_w % _CH == 0 and _CH % lanes == 0
    nch = b_per_w // _CH

    mesh = plsc.VectorSubcoreMesh(core_axis_name="c", subcore_axis_name="s")

    @functools.partial(
        pl.kernel,
        mesh=mesh,
        compiler_params=pltpu.CompilerParams(use_tc_tiling_on_sc=True),
        out_type=(
            jax.ShapeDtypeStruct((batch, emb), jnp.float32),
            jax.ShapeDtypeStruct((batch, emb), jnp.float32),
        ),
        scratch_types=[
            pltpu.VMEM((b_per_w,), jnp.int32),
            pltpu.VMEM((b_per_w,), jnp.int32),
            pltpu.VMEM((_CH, emb), jnp.float32),
            pltpu.VMEM((_CH, emb), jnp.float32),
            pltpu.SemaphoreType.DMA,
            pltpu.SemaphoreType.DMA,
            pltpu.SemaphoreType.DMA,
        ],
    )
    def k(user_hbm, item_hbm, ut_hbm, it_hbm, uout, iout,
          uidx, iidx, urows, irows, usem, isem, wsem):
        wid = lax.axis_index("s") * info.num_cores + lax.axis_index("c")
        base = wid * b_per_w
        pltpu.sync_copy(user_hbm.at[pl.ds(base, b_per_w)], uidx)
        pltpu.sync_copy(item_hbm.at[pl.ds(base, b_per_w)], iidx)

        def chunk(c, carry):
            def rbody(j, carry2):
                b = j * lanes
                uv = uidx[pl.ds(c * _CH + b, lanes)]
                iv = iidx[pl.ds(c * _CH + b, lanes)]
                for kk in range(lanes):
                    pltpu.async_copy(ut_hbm.at[uv[kk]],
                                     urows.at[b + kk], usem)
                    pltpu.async_copy(it_hbm.at[iv[kk]],
                                     irows.at[b + kk], isem)
                return carry2

            lax.fori_loop(0, _CH // lanes, rbody, 0)

            def rdrain(j, carry2):
                pltpu.make_async_copy(ut_hbm.at[0], urows.at[0],
                                      usem).wait()
                pltpu.make_async_copy(it_hbm.at[0], irows.at[0],
                                      isem).wait()
                return carry2

            lax.fori_loop(0, _CH, rdrain, 0)

            cbase = base + c * _CH
            ucp = pltpu.make_async_copy(
                urows, uout.at[pl.ds(cbase, _CH)], wsem)
            icp = pltpu.make_async_copy(
                irows, iout.at[pl.ds(cbase, _CH)], wsem)
            ucp.start()
            icp.start()
            ucp.wait()
            icp.wait()
            return carry

        lax.fori_loop(0, nch, chunk, 0)

    return k


def kernel(user, item, user_table, item_table):
    batch, = user.shape
    n_users, emb = user_table.shape
    n_items, _ = item_table.shape
    k = _build(batch, emb, n_users, n_items)
    return k(user, item, user_table, item_table)


# speedup vs baseline: 1.5708x; 1.0034x over previous
import functools

import jax
import jax.numpy as jnp
from jax import lax
from jax.experimental import pallas as pl
from jax.experimental.pallas import tpu as pltpu
from jax.experimental.pallas import tpu_sc as plsc

_CH = 128
_NSEM = 4


@functools.lru_cache(maxsize=None)
def _build(batch, emb, n_users, n_items):
    info = plsc.get_sparse_core_info()
    nw = info.num_cores * info.num_subcores
    lanes = info.num_lanes
    b_per_w = batch // nw
    assert b_per_w * nw == batch and b_per_w % _CH == 0 and _CH % lanes == 0
    nch = b_per_w // _CH

    mesh = plsc.VectorSubcoreMesh(core_axis_name="c", subcore_axis_name="s")

    @functools.partial(
        pl.kernel,
        mesh=mesh,
        compiler_params=pltpu.CompilerParams(use_tc_tiling_on_sc=True),
        out_type=(
            jax.ShapeDtypeStruct((batch, emb), jnp.float32),
            jax.ShapeDtypeStruct((batch, emb), jnp.float32),
        ),
        scratch_types=[
            pltpu.VMEM((b_per_w,), jnp.int32),
            pltpu.VMEM((b_per_w,), jnp.int32),
            pltpu.VMEM((_CH, emb), jnp.float32),
            pltpu.VMEM((_CH, emb), jnp.float32),
            [pltpu.SemaphoreType.DMA] * _NSEM,
            [pltpu.SemaphoreType.DMA] * _NSEM,
            pltpu.SemaphoreType.DMA,
        ],
    )
    def k(user_hbm, item_hbm, ut_hbm, it_hbm, uout, iout,
          uidx, iidx, urows, irows, usems, isems, wsem):
        wid = lax.axis_index("s") * info.num_cores + lax.axis_index("c")
        base = wid * b_per_w
        pltpu.sync_copy(user_hbm.at[pl.ds(base, b_per_w)], uidx)
        pltpu.sync_copy(item_hbm.at[pl.ds(base, b_per_w)], iidx)

        def chunk(c, carry):
            def rbody(j, carry2):
                b = j * lanes
                uv = uidx[pl.ds(c * _CH + b, lanes)]
                iv = iidx[pl.ds(c * _CH + b, lanes)]
                for kk in range(lanes):
                    pltpu.async_copy(ut_hbm.at[uv[kk]],
                                     urows.at[b + kk], usems[kk % _NSEM])
                    pltpu.async_copy(it_hbm.at[iv[kk]],
                                     irows.at[b + kk], isems[kk % _NSEM])
                return carry2

            lax.fori_loop(0, _CH // lanes, rbody, 0)

            def rdrain(j, carry2):
                for s in range(_NSEM):
                    pltpu.make_async_copy(ut_hbm.at[0], urows.at[0],
                                          usems[s]).wait()
                    pltpu.make_async_copy(it_hbm.at[0], irows.at[0],
                                          isems[s]).wait()
                return carry2

            lax.fori_loop(0, _CH // _NSEM, rdrain, 0)

            cbase = base + c * _CH
            ucp = pltpu.make_async_copy(
                urows, uout.at[pl.ds(cbase, _CH)], wsem)
            icp = pltpu.make_async_copy(
                irows, iout.at[pl.ds(cbase, _CH)], wsem)
            ucp.start()
            icp.start()
            ucp.wait()
            icp.wait()
            return carry

        lax.fori_loop(0, nch, chunk, 0)

    return k


def kernel(user, item, user_table, item_table):
    batch, = user.shape
    n_users, emb = user_table.shape
    n_items, _ = item_table.shape
    k = _build(batch, emb, n_users, n_items)
    return k(user, item, user_table, item_table)
